# Initial kernel scaffold; baseline (speedup 1.0000x reference)
import jax
import jax.numpy as jnp
from jax.experimental import pallas as pl


def kernel(row_x, token_x, t2r_edge_index, edge_attr_t2r, r2t_edge_index, edge_attr_r2t, W_t2r, b_t2r, W_r2t, b_r2t, g_row, beta_row, g_tok, beta_tok):
    raise NotImplementedError("write your pallas kernel here")


# speedup vs baseline: 2.7648x; 2.7648x over previous
import functools

import jax
import jax.numpy as jnp
from jax import lax
from jax.experimental import pallas as pl
from jax.experimental.pallas import tpu as pltpu
from jax.experimental.pallas import tpu_sc as plsc

N_NODE = 10000
N_PAD = 10240
E = 320000
HID = 128
EDIM = 16
PACK = HID // EDIM

NUM_CORES = 2
NUM_SUBCORES = 16
NUM_WORKERS = NUM_CORES * NUM_SUBCORES
KBLK = 128
FULL_BLKS = E // KBLK
BASE_BLKS = FULL_BLKS // NUM_WORKERS
EXTRA_BLKS = FULL_BLKS - BASE_BLKS * NUM_WORKERS
GKBLK = 80
GBLKS = (E // NUM_WORKERS) // GKBLK
ROWS_PER_TILE = N_PAD // NUM_SUBCORES
ZCH = 128
NZCH = ROWS_PER_TILE // ZCH
OUT_ROWS = N_PAD // PACK
TILE_WROWS = ROWS_PER_TILE // PACK

_sc_mesh = lambda: plsc.VectorSubcoreMesh(core_axis_name="c",
                                          subcore_axis_name="s")


def _fill_row_ids(zidx_v, base):
    for g in range(ZCH // 16):
        zidx_v[pl.ds(g * 16, 16)] = (jnp.arange(16, dtype=jnp.int32)
                                     + (base + g * 16))


def _sc_attr_cnt(dst, z, zw):
    @functools.partial(
        pl.kernel,
        out_type=jax.ShapeDtypeStruct((NUM_CORES * N_PAD, HID), jnp.float32),
        mesh=_sc_mesh(),
        scratch_types=[
            pltpu.VMEM((GKBLK,), jnp.int32),
            pltpu.VMEM((GKBLK, HID), jnp.float32),
            pltpu.VMEM_SHARED((N_PAD, HID), jnp.float32),
        ],
    )
    def k(dst_h, z_h, zw_h, sz_h, dst_v, rows_v, acc_s):
        c = lax.axis_index("c")
        s = lax.axis_index("s")

        rbase = s * ROWS_PER_TILE
        for zc in range(NZCH):
            pltpu.sync_copy(zw_h, acc_s.at[pl.ds(rbase + zc * ZCH, ZCH)])
        plsc.subcore_barrier()

        wid = s * NUM_CORES + c
        ebase = wid * (E // NUM_WORKERS)

        @pl.loop(0, GBLKS)
        def _(i):
            o = ebase + i * GKBLK
            pltpu.sync_copy(dst_h.at[pl.ds(o, GKBLK)], dst_v)
            pltpu.sync_copy(z_h.at[pl.ds(o, GKBLK)], rows_v)
            pltpu.sync_copy(rows_v, acc_s.at[dst_v], add=True)
        plsc.subcore_barrier()

        pltpu.sync_copy(acc_s.at[pl.ds(rbase, ROWS_PER_TILE)],
                        sz_h.at[pl.ds(c * N_PAD + rbase, ROWS_PER_TILE)])

    return k(dst, z, zw).reshape(NUM_CORES, N_PAD, HID)


def _sc_gather_sum(table, src, dst, zw):
    @functools.partial(
        pl.kernel,
        out_type=jax.ShapeDtypeStruct((NUM_CORES * N_PAD, HID), jnp.float32),
        mesh=_sc_mesh(),
        scratch_types=[
            pltpu.VMEM((GKBLK,), jnp.int32),
            pltpu.VMEM((GKBLK,), jnp.int32),
            pltpu.VMEM((GKBLK, HID), jnp.float32),
            pltpu.SemaphoreType.DMA,
            pltpu.VMEM_SHARED((N_PAD, HID), jnp.float32),
        ],
    )
    def k(table_h, src_h, dst_h, zw_h, sx_h,
          src_v, dst_v, rows_v, sem, acc_s):
        c = lax.axis_index("c")
        s = lax.axis_index("s")

        rbase = s * ROWS_PER_TILE
        for z in range(NZCH):
            pltpu.sync_copy(zw_h, acc_s.at[pl.ds(rbase + z * ZCH, ZCH)])
        plsc.subcore_barrier()

        wid = s * NUM_CORES + c
        ebase = wid * (E // NUM_WORKERS)

        @pl.loop(0, GBLKS)
        def _(i):
            o = ebase + i * GKBLK
            pltpu.sync_copy(src_h.at[pl.ds(o, GKBLK)], src_v)
            pltpu.sync_copy(dst_h.at[pl.ds(o, GKBLK)], dst_v)
            pltpu.async_copy(table_h.at[src_v], rows_v, sem).wait()
            pltpu.sync_copy(rows_v, acc_s.at[dst_v], add=True)
        plsc.subcore_barrier()

        pltpu.sync_copy(acc_s.at[pl.ds(rbase, ROWS_PER_TILE)],
                        sx_h.at[pl.ds(c * N_PAD + rbase, ROWS_PER_TILE)])

    return k(table, src, dst, zw).reshape(NUM_CORES, N_PAD, HID)


_TC_ROWS = 1280


def _tc_body(sx_ref, sz_ref, x_ref, wx_ref, we_ref, b_ref,
             g_ref, beta_ref, o_ref):
    sx = sx_ref[0] + sx_ref[1]
    sz = sz_ref[0] + sz_ref[1]
    sa = sz[:, 0:EDIM]
    cnt = sz[:, EDIM:EDIM + 1]
    agg = (jnp.dot(sx, wx_ref[...], preferred_element_type=jnp.float32)
           + jnp.dot(sa, we_ref[...], preferred_element_type=jnp.float32)
           + cnt * b_ref[...])
    msg = agg / jnp.maximum(cnt, 1.0)
    h = x_ref[...] + msg
    mu = jnp.mean(h, axis=1, keepdims=True)
    var = jnp.mean((h - mu) ** 2, axis=1, keepdims=True)
    o_ref[...] = (h - mu) * lax.rsqrt(var + 1e-5) * g_ref[...] + beta_ref[...]


def _tc_epilogue(sx, sz, x, wx, we, b, g, beta):
    grid = (N_PAD // _TC_ROWS,)
    return pl.pallas_call(
        _tc_body,
        grid=grid,
        in_specs=[
            pl.BlockSpec((NUM_CORES, _TC_ROWS, HID), lambda i: (0, i, 0)),
            pl.BlockSpec((NUM_CORES, _TC_ROWS, HID), lambda i: (0, i, 0)),
            pl.BlockSpec((_TC_ROWS, HID), lambda i: (i, 0)),
            pl.BlockSpec((HID, HID), lambda i: (0, 0)),
            pl.BlockSpec((EDIM, HID), lambda i: (0, 0)),
            pl.BlockSpec((1, HID), lambda i: (0, 0)),
            pl.BlockSpec((1, HID), lambda i: (0, 0)),
            pl.BlockSpec((1, HID), lambda i: (0, 0)),
        ],
        out_specs=pl.BlockSpec((_TC_ROWS, HID), lambda i: (i, 0)),
        out_shape=jax.ShapeDtypeStruct((N_NODE, HID), jnp.float32),
    )(sx, sz, x, wx, we, b, g, beta)


def kernel(row_x, token_x, t2r_edge_index, edge_attr_t2r, r2t_edge_index,
           edge_attr_r2t, W_t2r, b_t2r, W_r2t, b_r2t, g_row, beta_row,
           g_tok, beta_tok):
    src_t2r = t2r_edge_index[0].astype(jnp.int32)
    dst_t2r = t2r_edge_index[1].astype(jnp.int32)
    src_r2t = r2t_edge_index[0].astype(jnp.int32)
    dst_r2t = r2t_edge_index[1].astype(jnp.int32)

    zw = jnp.zeros((ZCH, HID), jnp.float32)
    pad = jnp.concatenate(
        [jnp.ones((E, 1), jnp.float32),
         jnp.zeros((E, HID - EDIM - 1), jnp.float32)], axis=1)
    z1 = jnp.concatenate([edge_attr_t2r, pad], axis=1)
    z2 = jnp.concatenate([edge_attr_r2t, pad], axis=1)

    sz1 = _sc_attr_cnt(dst_t2r, z1, zw)
    sz2 = _sc_attr_cnt(dst_r2t, z2, zw)

    sx1 = _sc_gather_sum(token_x, src_t2r, dst_t2r, zw)
    row_new = _tc_epilogue(sx1, sz1, row_x,
                           W_t2r[:HID], W_t2r[HID:], b_t2r.reshape(1, HID),
                           g_row.reshape(1, HID), beta_row.reshape(1, HID))

    sx2 = _sc_gather_sum(row_new, src_r2t, dst_r2t, zw)
    tok_new = _tc_epilogue(sx2, sz2, token_x,
                           W_r2t[:HID], W_r2t[HID:], b_r2t.reshape(1, HID),
                           g_tok.reshape(1, HID), beta_tok.reshape(1, HID))

    return (row_new, tok_new)


# speedup vs baseline: 3.6813x; 1.3315x over previous
import functools

import jax
import jax.numpy as jnp
from jax import lax
from jax.experimental import pallas as pl
from jax.experimental.pallas import tpu as pltpu
from jax.experimental.pallas import tpu_sc as plsc

N_NODE = 10000
N_PAD = 10240
E = 320000
HID = 128
EDIM = 16
PACK = HID // EDIM

NUM_CORES = 2
NUM_SUBCORES = 16
NUM_WORKERS = NUM_CORES * NUM_SUBCORES
KBLK = 128
FULL_BLKS = E // KBLK
BASE_BLKS = FULL_BLKS // NUM_WORKERS
EXTRA_BLKS = FULL_BLKS - BASE_BLKS * NUM_WORKERS
GKBLK = 80
GBLKS = (E // NUM_WORKERS) // GKBLK
ROWS_PER_TILE = N_PAD // NUM_SUBCORES
ZCH = 128
NZCH = ROWS_PER_TILE // ZCH
OUT_ROWS = N_PAD // PACK
TILE_WROWS = ROWS_PER_TILE // PACK

_sc_mesh = lambda: plsc.VectorSubcoreMesh(core_axis_name="c",
                                          subcore_axis_name="s")


def _fill_row_ids(zidx_v, base):
    for g in range(ZCH // 16):
        zidx_v[pl.ds(g * 16, 16)] = (jnp.arange(16, dtype=jnp.int32)
                                     + (base + g * 16))


def _sc_attr_cnt(dst, z, zw):
    @functools.partial(
        pl.kernel,
        out_type=jax.ShapeDtypeStruct((NUM_CORES * N_PAD, HID), jnp.float32),
        mesh=_sc_mesh(),
        scratch_types=[
            pltpu.VMEM((2, GKBLK), jnp.int32),
            pltpu.VMEM((2, GKBLK, HID), jnp.float32),
            pltpu.SemaphoreType.DMA,
            pltpu.SemaphoreType.DMA,
            pltpu.VMEM_SHARED((N_PAD, HID), jnp.float32),
        ],
    )
    def k(dst_h, z_h, zw_h, sz_h, dst_v, rows_v, sem0, sem1, acc_s):
        c = lax.axis_index("c")
        s = lax.axis_index("s")
        sems = (sem0, sem1)

        rbase = s * ROWS_PER_TILE
        for zc in range(NZCH):
            pltpu.sync_copy(zw_h, acc_s.at[pl.ds(rbase + zc * ZCH, ZCH)])
        plsc.subcore_barrier()

        wid = s * NUM_CORES + c
        ebase = wid * (E // NUM_WORKERS)

        def fire(i, bb):
            o = ebase + i * GKBLK
            pltpu.sync_copy(dst_h.at[pl.ds(o, GKBLK)], dst_v.at[bb])
            return pltpu.async_copy(z_h.at[pl.ds(o, GKBLK)], rows_v.at[bb],
                                    sems[bb])

        def drain(bb, cp):
            cp.wait()
            pltpu.sync_copy(rows_v.at[bb], acc_s.at[dst_v.at[bb]], add=True)

        @pl.loop(0, GBLKS // 2)
        def _(i2):
            cp0 = fire(i2 * 2, 0)
            cp1 = fire(i2 * 2 + 1, 1)
            drain(0, cp0)
            drain(1, cp1)
        for i in range(GBLKS - (GBLKS % 2), GBLKS):
            drain(0, fire(i, 0))
        plsc.subcore_barrier()

        pltpu.sync_copy(acc_s.at[pl.ds(rbase, ROWS_PER_TILE)],
                        sz_h.at[pl.ds(c * N_PAD + rbase, ROWS_PER_TILE)])

    return k(dst, z, zw).reshape(NUM_CORES, N_PAD, HID)


def _sc_gather_sum(table, src, dst, zw):
    @functools.partial(
        pl.kernel,
        out_type=jax.ShapeDtypeStruct((NUM_CORES * N_PAD, HID), jnp.float32),
        mesh=_sc_mesh(),
        scratch_types=[
            pltpu.VMEM((2, GKBLK), jnp.int32),
            pltpu.VMEM((2, GKBLK), jnp.int32),
            pltpu.VMEM((2, GKBLK, HID), jnp.float32),
            pltpu.SemaphoreType.DMA,
            pltpu.SemaphoreType.DMA,
            pltpu.VMEM_SHARED((N_PAD, HID), jnp.float32),
        ],
    )
    def k(table_h, src_h, dst_h, zw_h, sx_h,
          src_v, dst_v, rows_v, sem0, sem1, acc_s):
        c = lax.axis_index("c")
        s = lax.axis_index("s")
        sems = (sem0, sem1)

        rbase = s * ROWS_PER_TILE
        for z in range(NZCH):
            pltpu.sync_copy(zw_h, acc_s.at[pl.ds(rbase + z * ZCH, ZCH)])
        plsc.subcore_barrier()

        wid = s * NUM_CORES + c
        ebase = wid * (E // NUM_WORKERS)

        def fire(i, bb):
            o = ebase + i * GKBLK
            pltpu.sync_copy(src_h.at[pl.ds(o, GKBLK)], src_v.at[bb])
            pltpu.sync_copy(dst_h.at[pl.ds(o, GKBLK)], dst_v.at[bb])
            return pltpu.async_copy(table_h.at[src_v.at[bb]], rows_v.at[bb],
                                    sems[bb])

        def drain(bb, cp):
            cp.wait()
            pltpu.sync_copy(rows_v.at[bb], acc_s.at[dst_v.at[bb]], add=True)

        @pl.loop(0, GBLKS // 2)
        def _(i2):
            cp0 = fire(i2 * 2, 0)
            cp1 = fire(i2 * 2 + 1, 1)
            drain(0, cp0)
            drain(1, cp1)
        for i in range(GBLKS - (GBLKS % 2), GBLKS):
            drain(0, fire(i, 0))
        plsc.subcore_barrier()

        pltpu.sync_copy(acc_s.at[pl.ds(rbase, ROWS_PER_TILE)],
                        sx_h.at[pl.ds(c * N_PAD + rbase, ROWS_PER_TILE)])

    return k(table, src, dst, zw).reshape(NUM_CORES, N_PAD, HID)


_TC_ROWS = 1280


def _tc_body(sx_ref, sz_ref, x_ref, wx_ref, we_ref, b_ref,
             g_ref, beta_ref, o_ref):
    sx = sx_ref[0] + sx_ref[1]
    sz = sz_ref[0] + sz_ref[1]
    sa = sz[:, 0:EDIM]
    cnt = sz[:, EDIM:EDIM + 1]
    agg = (jnp.dot(sx, wx_ref[...], preferred_element_type=jnp.float32)
           + jnp.dot(sa, we_ref[...], preferred_element_type=jnp.float32)
           + cnt * b_ref[...])
    msg = agg / jnp.maximum(cnt, 1.0)
    h = x_ref[...] + msg
    mu = jnp.mean(h, axis=1, keepdims=True)
    var = jnp.mean((h - mu) ** 2, axis=1, keepdims=True)
    o_ref[...] = (h - mu) * lax.rsqrt(var + 1e-5) * g_ref[...] + beta_ref[...]


def _tc_epilogue(sx, sz, x, wx, we, b, g, beta):
    grid = (N_PAD // _TC_ROWS,)
    return pl.pallas_call(
        _tc_body,
        grid=grid,
        in_specs=[
            pl.BlockSpec((NUM_CORES, _TC_ROWS, HID), lambda i: (0, i, 0)),
            pl.BlockSpec((NUM_CORES, _TC_ROWS, HID), lambda i: (0, i, 0)),
            pl.BlockSpec((_TC_ROWS, HID), lambda i: (i, 0)),
            pl.BlockSpec((HID, HID), lambda i: (0, 0)),
            pl.BlockSpec((EDIM, HID), lambda i: (0, 0)),
            pl.BlockSpec((1, HID), lambda i: (0, 0)),
            pl.BlockSpec((1, HID), lambda i: (0, 0)),
            pl.BlockSpec((1, HID), lambda i: (0, 0)),
        ],
        out_specs=pl.BlockSpec((_TC_ROWS, HID), lambda i: (i, 0)),
        out_shape=jax.ShapeDtypeStruct((N_NODE, HID), jnp.float32),
    )(sx, sz, x, wx, we, b, g, beta)


def kernel(row_x, token_x, t2r_edge_index, edge_attr_t2r, r2t_edge_index,
           edge_attr_r2t, W_t2r, b_t2r, W_r2t, b_r2t, g_row, beta_row,
           g_tok, beta_tok):
    src_t2r = t2r_edge_index[0].astype(jnp.int32)
    dst_t2r = t2r_edge_index[1].astype(jnp.int32)
    src_r2t = r2t_edge_index[0].astype(jnp.int32)
    dst_r2t = r2t_edge_index[1].astype(jnp.int32)

    zw = jnp.zeros((ZCH, HID), jnp.float32)
    pad = jnp.concatenate(
        [jnp.ones((E, 1), jnp.float32),
         jnp.zeros((E, HID - EDIM - 1), jnp.float32)], axis=1)
    z1 = jnp.concatenate([edge_attr_t2r, pad], axis=1)
    z2 = jnp.concatenate([edge_attr_r2t, pad], axis=1)

    sz1 = _sc_attr_cnt(dst_t2r, z1, zw)
    sz2 = _sc_attr_cnt(dst_r2t, z2, zw)

    sx1 = _sc_gather_sum(token_x, src_t2r, dst_t2r, zw)
    row_new = _tc_epilogue(sx1, sz1, row_x,
                           W_t2r[:HID], W_t2r[HID:], b_t2r.reshape(1, HID),
                           g_row.reshape(1, HID), beta_row.reshape(1, HID))

    sx2 = _sc_gather_sum(row_new, src_r2t, dst_r2t, zw)
    tok_new = _tc_epilogue(sx2, sz2, token_x,
                           W_r2t[:HID], W_r2t[HID:], b_r2t.reshape(1, HID),
                           g_tok.reshape(1, HID), beta_tok.reshape(1, HID))

    return (row_new, tok_new)


# speedup vs baseline: 4.2593x; 1.1570x over previous
import functools

import jax
import jax.numpy as jnp
from jax import lax
from jax.experimental import pallas as pl
from jax.experimental.pallas import tpu as pltpu
from jax.experimental.pallas import tpu_sc as plsc

N_NODE = 10000
N_PAD = 10240
E = 320000
HID = 128
EDIM = 16
PACK = HID // EDIM

NUM_CORES = 2
NUM_SUBCORES = 16
NUM_WORKERS = NUM_CORES * NUM_SUBCORES
KBLK = 128
FULL_BLKS = E // KBLK
BASE_BLKS = FULL_BLKS // NUM_WORKERS
EXTRA_BLKS = FULL_BLKS - BASE_BLKS * NUM_WORKERS
GKBLK = 80
GBLKS = (E // NUM_WORKERS) // GKBLK
ROWS_PER_TILE = N_PAD // NUM_SUBCORES
ZCH = 128
NZCH = ROWS_PER_TILE // ZCH
OUT_ROWS = N_PAD // PACK
TILE_WROWS = ROWS_PER_TILE // PACK

_sc_mesh = lambda: plsc.VectorSubcoreMesh(core_axis_name="c",
                                          subcore_axis_name="s")


def _fill_row_ids(zidx_v, base):
    for g in range(ZCH // 16):
        zidx_v[pl.ds(g * 16, 16)] = (jnp.arange(16, dtype=jnp.int32)
                                     + (base + g * 16))


def _sc_attr_cnt(dst, z, zw):
    @functools.partial(
        pl.kernel,
        out_type=jax.ShapeDtypeStruct((NUM_CORES * N_PAD, HID), jnp.float32),
        mesh=_sc_mesh(),
        scratch_types=[
            pltpu.VMEM((4, GKBLK), jnp.int32),
            pltpu.VMEM((4, GKBLK, HID), jnp.float32),
            pltpu.SemaphoreType.DMA,
            pltpu.SemaphoreType.DMA,
            pltpu.SemaphoreType.DMA,
            pltpu.SemaphoreType.DMA,
            pltpu.VMEM_SHARED((N_PAD, HID), jnp.float32),
        ],
    )
    def k(dst_h, z_h, zw_h, sz_h, dst_v, rows_v, sem0, sem1, sem2, sem3,
          acc_s):
        c = lax.axis_index("c")
        s = lax.axis_index("s")
        sems = (sem0, sem1, sem2, sem3)

        rbase = s * ROWS_PER_TILE
        for zc in range(NZCH):
            pltpu.sync_copy(zw_h, acc_s.at[pl.ds(rbase + zc * ZCH, ZCH)])
        plsc.subcore_barrier()

        wid = s * NUM_CORES + c
        ebase = wid * (E // NUM_WORKERS)

        def fire(i, bb):
            o = ebase + i * GKBLK
            pltpu.sync_copy(dst_h.at[pl.ds(o, GKBLK)], dst_v.at[bb])
            return pltpu.async_copy(z_h.at[pl.ds(o, GKBLK)], rows_v.at[bb],
                                    sems[bb])

        def drain(bb, cp):
            cp.wait()
            pltpu.sync_copy(rows_v.at[bb], acc_s.at[dst_v.at[bb]], add=True)

        @pl.loop(0, GBLKS // 4)
        def _(i4):
            cps = [fire(i4 * 4 + bb, bb) for bb in range(4)]
            for bb in range(4):
                drain(bb, cps[bb])
        for i in range(GBLKS - (GBLKS % 4), GBLKS):
            drain(0, fire(i, 0))
        plsc.subcore_barrier()

        pltpu.sync_copy(acc_s.at[pl.ds(rbase, ROWS_PER_TILE)],
                        sz_h.at[pl.ds(c * N_PAD + rbase, ROWS_PER_TILE)])

    return k(dst, z, zw).reshape(NUM_CORES, N_PAD, HID)


def _sc_gather_sum(table, src, dst, zw):
    @functools.partial(
        pl.kernel,
        out_type=jax.ShapeDtypeStruct((NUM_CORES * N_PAD, HID), jnp.float32),
        mesh=_sc_mesh(),
        scratch_types=[
            pltpu.VMEM((4, GKBLK), jnp.int32),
            pltpu.VMEM((4, GKBLK), jnp.int32),
            pltpu.VMEM((4, GKBLK, HID), jnp.float32),
            pltpu.SemaphoreType.DMA,
            pltpu.SemaphoreType.DMA,
            pltpu.SemaphoreType.DMA,
            pltpu.SemaphoreType.DMA,
            pltpu.VMEM_SHARED((N_PAD, HID), jnp.float32),
        ],
    )
    def k(table_h, src_h, dst_h, zw_h, sx_h,
          src_v, dst_v, rows_v, sem0, sem1, sem2, sem3, acc_s):
        c = lax.axis_index("c")
        s = lax.axis_index("s")
        sems = (sem0, sem1, sem2, sem3)

        rbase = s * ROWS_PER_TILE
        for z in range(NZCH):
            pltpu.sync_copy(zw_h, acc_s.at[pl.ds(rbase + z * ZCH, ZCH)])
        plsc.subcore_barrier()

        wid = s * NUM_CORES + c
        ebase = wid * (E // NUM_WORKERS)

        def fire(i, bb):
            o = ebase + i * GKBLK
            pltpu.sync_copy(src_h.at[pl.ds(o, GKBLK)], src_v.at[bb])
            pltpu.sync_copy(dst_h.at[pl.ds(o, GKBLK)], dst_v.at[bb])
            return pltpu.async_copy(table_h.at[src_v.at[bb]], rows_v.at[bb],
                                    sems[bb])

        def drain(bb, cp):
            cp.wait()
            pltpu.sync_copy(rows_v.at[bb], acc_s.at[dst_v.at[bb]], add=True)

        @pl.loop(0, GBLKS // 4)
        def _(i4):
            cps = [fire(i4 * 4 + bb, bb) for bb in range(4)]
            for bb in range(4):
                drain(bb, cps[bb])
        for i in range(GBLKS - (GBLKS % 4), GBLKS):
            drain(0, fire(i, 0))
        plsc.subcore_barrier()

        pltpu.sync_copy(acc_s.at[pl.ds(rbase, ROWS_PER_TILE)],
                        sx_h.at[pl.ds(c * N_PAD + rbase, ROWS_PER_TILE)])

    return k(table, src, dst, zw).reshape(NUM_CORES, N_PAD, HID)


_TC_ROWS = 1280


def _tc_body(sx_ref, sz_ref, x_ref, wx_ref, we_ref, b_ref,
             g_ref, beta_ref, o_ref):
    sx = sx_ref[0] + sx_ref[1]
    sz = sz_ref[0] + sz_ref[1]
    sa = sz[:, 0:EDIM]
    cnt = sz[:, EDIM:EDIM + 1]
    agg = (jnp.dot(sx, wx_ref[...], preferred_element_type=jnp.float32)
           + jnp.dot(sa, we_ref[...], preferred_element_type=jnp.float32)
           + cnt * b_ref[...])
    msg = agg / jnp.maximum(cnt, 1.0)
    h = x_ref[...] + msg
    mu = jnp.mean(h, axis=1, keepdims=True)
    var = jnp.mean((h - mu) ** 2, axis=1, keepdims=True)
    o_ref[...] = (h - mu) * lax.rsqrt(var + 1e-5) * g_ref[...] + beta_ref[...]


def _tc_epilogue(sx, sz, x, wx, we, b, g, beta):
    grid = (N_PAD // _TC_ROWS,)
    return pl.pallas_call(
        _tc_body,
        grid=grid,
        in_specs=[
            pl.BlockSpec((NUM_CORES, _TC_ROWS, HID), lambda i: (0, i, 0)),
            pl.BlockSpec((NUM_CORES, _TC_ROWS, HID), lambda i: (0, i, 0)),
            pl.BlockSpec((_TC_ROWS, HID), lambda i: (i, 0)),
            pl.BlockSpec((HID, HID), lambda i: (0, 0)),
            pl.BlockSpec((EDIM, HID), lambda i: (0, 0)),
            pl.BlockSpec((1, HID), lambda i: (0, 0)),
            pl.BlockSpec((1, HID), lambda i: (0, 0)),
            pl.BlockSpec((1, HID), lambda i: (0, 0)),
        ],
        out_specs=pl.BlockSpec((_TC_ROWS, HID), lambda i: (i, 0)),
        out_shape=jax.ShapeDtypeStruct((N_NODE, HID), jnp.float32),
    )(sx, sz, x, wx, we, b, g, beta)


def kernel(row_x, token_x, t2r_edge_index, edge_attr_t2r, r2t_edge_index,
           edge_attr_r2t, W_t2r, b_t2r, W_r2t, b_r2t, g_row, beta_row,
           g_tok, beta_tok):
    src_t2r = t2r_edge_index[0].astype(jnp.int32)
    dst_t2r = t2r_edge_index[1].astype(jnp.int32)
    src_r2t = r2t_edge_index[0].astype(jnp.int32)
    dst_r2t = r2t_edge_index[1].astype(jnp.int32)

    zw = jnp.zeros((ZCH, HID), jnp.float32)
    pad = jnp.concatenate(
        [jnp.ones((E, 1), jnp.float32),
         jnp.zeros((E, HID - EDIM - 1), jnp.float32)], axis=1)
    z1 = jnp.concatenate([edge_attr_t2r, pad], axis=1)
    z2 = jnp.concatenate([edge_attr_r2t, pad], axis=1)

    sz1 = _sc_attr_cnt(dst_t2r, z1, zw)
    sz2 = _sc_attr_cnt(dst_r2t, z2, zw)

    sx1 = _sc_gather_sum(token_x, src_t2r, dst_t2r, zw)
    row_new = _tc_epilogue(sx1, sz1, row_x,
                           W_t2r[:HID], W_t2r[HID:], b_t2r.reshape(1, HID),
                           g_row.reshape(1, HID), beta_row.reshape(1, HID))

    sx2 = _sc_gather_sum(row_new, src_r2t, dst_r2t, zw)
    tok_new = _tc_epilogue(sx2, sz2, token_x,
                           W_r2t[:HID], W_r2t[HID:], b_r2t.reshape(1, HID),
                           g_tok.reshape(1, HID), beta_tok.reshape(1, HID))

    return (row_new, tok_new)


# speedup vs baseline: 4.2610x; 1.0004x over previous
import functools

import jax
import jax.numpy as jnp
from jax import lax
from jax.experimental import pallas as pl
from jax.experimental.pallas import tpu as pltpu
from jax.experimental.pallas import tpu_sc as plsc

N_NODE = 10000
N_PAD = 10240
E = 320000
HID = 128
EDIM = 16
NUM_CORES = 2
NUM_SUBCORES = 16
NUM_WORKERS = NUM_CORES * NUM_SUBCORES
GKBLK = 80
GBLKS = (E // NUM_WORKERS) // GKBLK
ROWS_PER_TILE = N_PAD // NUM_SUBCORES
ZCH = 128
NZCH = ROWS_PER_TILE // ZCH

_sc_mesh = lambda: plsc.VectorSubcoreMesh(core_axis_name="c",
                                          subcore_axis_name="s")


def _sc_attr_cnt(dst, z, zw):
    @functools.partial(
        pl.kernel,
        out_type=jax.ShapeDtypeStruct((NUM_CORES * N_PAD, HID), jnp.float32),
        mesh=_sc_mesh(),
        scratch_types=[
            pltpu.VMEM((4, GKBLK), jnp.int32),
            pltpu.VMEM((4, GKBLK, HID), jnp.float32),
            pltpu.SemaphoreType.DMA,
            pltpu.SemaphoreType.DMA,
            pltpu.SemaphoreType.DMA,
            pltpu.SemaphoreType.DMA,
            pltpu.VMEM_SHARED((N_PAD, HID), jnp.float32),
        ],
    )
    def k(dst_h, z_h, zw_h, sz_h, dst_v, rows_v, sem0, sem1, sem2, sem3,
          acc_s):
        c = lax.axis_index("c")
        s = lax.axis_index("s")
        sems = (sem0, sem1, sem2, sem3)

        rbase = s * ROWS_PER_TILE
        for zc in range(NZCH):
            pltpu.sync_copy(zw_h, acc_s.at[pl.ds(rbase + zc * ZCH, ZCH)])
        plsc.subcore_barrier()

        wid = s * NUM_CORES + c
        ebase = wid * (E // NUM_WORKERS)

        def fire(i, bb):
            o = ebase + i * GKBLK
            pltpu.sync_copy(dst_h.at[pl.ds(o, GKBLK)], dst_v.at[bb])
            return pltpu.async_copy(z_h.at[pl.ds(o, GKBLK)], rows_v.at[bb],
                                    sems[bb])

        def drain(bb, cp):
            cp.wait()
            pltpu.sync_copy(rows_v.at[bb], acc_s.at[dst_v.at[bb]], add=True)

        @pl.loop(0, GBLKS // 4)
        def _(i4):
            cps = [fire(i4 * 4 + bb, bb) for bb in range(4)]
            for bb in range(4):
                drain(bb, cps[bb])
        for i in range(GBLKS - (GBLKS % 4), GBLKS):
            drain(0, fire(i, 0))
        plsc.subcore_barrier()

        pltpu.sync_copy(acc_s.at[pl.ds(rbase, ROWS_PER_TILE)],
                        sz_h.at[pl.ds(c * N_PAD + rbase, ROWS_PER_TILE)])

    return k(dst, z, zw).reshape(NUM_CORES, N_PAD, HID)


def _sc_gather_sum(table, src, dst, zw):
    @functools.partial(
        pl.kernel,
        out_type=jax.ShapeDtypeStruct((NUM_CORES * N_PAD, HID), jnp.float32),
        mesh=_sc_mesh(),
        scratch_types=[
            pltpu.VMEM((4, GKBLK), jnp.int32),
            pltpu.VMEM((4, GKBLK), jnp.int32),
            pltpu.VMEM((4, GKBLK, HID), jnp.float32),
            pltpu.SemaphoreType.DMA,
            pltpu.SemaphoreType.DMA,
            pltpu.SemaphoreType.DMA,
            pltpu.SemaphoreType.DMA,
            pltpu.VMEM_SHARED((N_PAD, HID), jnp.float32),
        ],
    )
    def k(table_h, src_h, dst_h, zw_h, sx_h,
          src_v, dst_v, rows_v, sem0, sem1, sem2, sem3, acc_s):
        c = lax.axis_index("c")
        s = lax.axis_index("s")
        sems = (sem0, sem1, sem2, sem3)

        rbase = s * ROWS_PER_TILE
        for z in range(NZCH):
            pltpu.sync_copy(zw_h, acc_s.at[pl.ds(rbase + z * ZCH, ZCH)])
        plsc.subcore_barrier()

        wid = s * NUM_CORES + c
        ebase = wid * (E // NUM_WORKERS)

        def fire(i, bb):
            o = ebase + i * GKBLK
            pltpu.sync_copy(src_h.at[pl.ds(o, GKBLK)], src_v.at[bb])
            pltpu.sync_copy(dst_h.at[pl.ds(o, GKBLK)], dst_v.at[bb])
            return pltpu.async_copy(table_h.at[src_v.at[bb]], rows_v.at[bb],
                                    sems[bb])

        def drain(bb, cp):
            cp.wait()
            pltpu.sync_copy(rows_v.at[bb], acc_s.at[dst_v.at[bb]], add=True)

        @pl.loop(0, GBLKS // 4)
        def _(i4):
            cps = [fire(i4 * 4 + bb, bb) for bb in range(4)]
            for bb in range(4):
                drain(bb, cps[bb])
        for i in range(GBLKS - (GBLKS % 4), GBLKS):
            drain(0, fire(i, 0))
        plsc.subcore_barrier()

        pltpu.sync_copy(acc_s.at[pl.ds(rbase, ROWS_PER_TILE)],
                        sx_h.at[pl.ds(c * N_PAD + rbase, ROWS_PER_TILE)])

    return k(table, src, dst, zw).reshape(NUM_CORES, N_PAD, HID)


_TC_ROWS = 1280


def _tc_body(sx_ref, sz_ref, x_ref, wx_ref, we_ref, b_ref,
             g_ref, beta_ref, o_ref):
    sx = sx_ref[0] + sx_ref[1]
    sz = sz_ref[0] + sz_ref[1]
    sa = sz[:, 0:EDIM]
    cnt = sz[:, EDIM:EDIM + 1]
    agg = (jnp.dot(sx, wx_ref[...], preferred_element_type=jnp.float32)
           + jnp.dot(sa, we_ref[...], preferred_element_type=jnp.float32)
           + cnt * b_ref[...])
    msg = agg / jnp.maximum(cnt, 1.0)
    h = x_ref[...] + msg
    mu = jnp.mean(h, axis=1, keepdims=True)
    var = jnp.mean((h - mu) ** 2, axis=1, keepdims=True)
    o_ref[...] = (h - mu) * lax.rsqrt(var + 1e-5) * g_ref[...] + beta_ref[...]


def _tc_epilogue(sx, sz, x, wx, we, b, g, beta):
    grid = (N_PAD // _TC_ROWS,)
    return pl.pallas_call(
        _tc_body,
        grid=grid,
        in_specs=[
            pl.BlockSpec((NUM_CORES, _TC_ROWS, HID), lambda i: (0, i, 0)),
            pl.BlockSpec((NUM_CORES, _TC_ROWS, HID), lambda i: (0, i, 0)),
            pl.BlockSpec((_TC_ROWS, HID), lambda i: (i, 0)),
            pl.BlockSpec((HID, HID), lambda i: (0, 0)),
            pl.BlockSpec((EDIM, HID), lambda i: (0, 0)),
            pl.BlockSpec((1, HID), lambda i: (0, 0)),
            pl.BlockSpec((1, HID), lambda i: (0, 0)),
            pl.BlockSpec((1, HID), lambda i: (0, 0)),
        ],
        out_specs=pl.BlockSpec((_TC_ROWS, HID), lambda i: (i, 0)),
        out_shape=jax.ShapeDtypeStruct((N_NODE, HID), jnp.float32),
    )(sx, sz, x, wx, we, b, g, beta)


def kernel(row_x, token_x, t2r_edge_index, edge_attr_t2r, r2t_edge_index,
           edge_attr_r2t, W_t2r, b_t2r, W_r2t, b_r2t, g_row, beta_row,
           g_tok, beta_tok):
    src_t2r = t2r_edge_index[0].astype(jnp.int32)
    dst_t2r = t2r_edge_index[1].astype(jnp.int32)
    src_r2t = r2t_edge_index[0].astype(jnp.int32)
    dst_r2t = r2t_edge_index[1].astype(jnp.int32)

    zw = jnp.zeros((ZCH, HID), jnp.float32)
    pad = jnp.concatenate(
        [jnp.ones((E, 1), jnp.float32),
         jnp.zeros((E, HID - EDIM - 1), jnp.float32)], axis=1)
    z1 = jnp.concatenate([edge_attr_t2r, pad], axis=1)
    z2 = jnp.concatenate([edge_attr_r2t, pad], axis=1)

    sz1 = _sc_attr_cnt(dst_t2r, z1, zw)
    sz2 = _sc_attr_cnt(dst_r2t, z2, zw)

    sx1 = _sc_gather_sum(token_x, src_t2r, dst_t2r, zw)
    row_new = _tc_epilogue(sx1, sz1, row_x,
                           W_t2r[:HID], W_t2r[HID:], b_t2r.reshape(1, HID),
                           g_row.reshape(1, HID), beta_row.reshape(1, HID))

    sx2 = _sc_gather_sum(row_new, src_r2t, dst_r2t, zw)
    tok_new = _tc_epilogue(sx2, sz2, token_x,
                           W_r2t[:HID], W_r2t[HID:], b_r2t.reshape(1, HID),
                           g_tok.reshape(1, HID), beta_tok.reshape(1, HID))

    return (row_new, tok_new)
